# baseline (speedup 1.0000x reference)
import functools

import jax
import jax.numpy as jnp
from jax import lax
from jax.experimental import pallas as pl
from jax.experimental.pallas import tpu as pltpu
from jax.experimental.pallas import tpu_sc as plsc

D = 64
MARGIN = 2.0
NC = 2
NS = 16
NW = NC * NS
CH = 128


@functools.partial(jax.jit, static_argnames=("B",))
def _transe_sc(heads, relations, tails, h_hat, t_hat, ent, rel, *, B):
    CB = B // NW
    NCH = CB // CH

    mesh = plsc.VectorSubcoreMesh(core_axis_name="c", subcore_axis_name="s",
                                  num_cores=NC, num_subcores=NS)
    row_buf = pltpu.VMEM((2, CH, D), jnp.float32)
    idx_buf = pltpu.VMEM((NCH, CH), jnp.int32)

    @functools.partial(
        pl.kernel,
        out_type=jax.ShapeDtypeStruct((NW, CB), jnp.float32),
        mesh=mesh,
        scratch_types=[
            idx_buf, idx_buf, idx_buf, idx_buf, idx_buf,
            row_buf, row_buf, row_buf, row_buf, row_buf,
            pltpu.VMEM((CB,), jnp.float32),
            pltpu.SemaphoreType.DMA,
            pltpu.SemaphoreType.DMA,
        ],
        compiler_params=pltpu.CompilerParams(use_tc_tiling_on_sc=False),
    )
    def k(h_idx_hbm, r_idx_hbm, t_idx_hbm, hh_idx_hbm, th_idx_hbm,
          ent_hbm, rel_hbm, out_hbm,
          hi, ri, ti, hhi, thi, hv, rv, tv, hhv, thv, out_v, sem0, sem1):
        wid = lax.axis_index("s") * NC + lax.axis_index("c")

        pltpu.sync_copy(h_idx_hbm.at[wid], hi)
        pltpu.sync_copy(r_idx_hbm.at[wid], ri)
        pltpu.sync_copy(t_idx_hbm.at[wid], ti)
        pltpu.sync_copy(hh_idx_hbm.at[wid], hhi)
        pltpu.sync_copy(th_idx_hbm.at[wid], thi)

        sems = (sem0, sem1)

        def fire(kk):
            slot = kk % 2
            sem = sems[slot]
            return [
                pltpu.async_copy(ent_hbm.at[hi.at[kk]], hv.at[slot], sem),
                pltpu.async_copy(rel_hbm.at[ri.at[kk]], rv.at[slot], sem),
                pltpu.async_copy(ent_hbm.at[ti.at[kk]], tv.at[slot], sem),
                pltpu.async_copy(ent_hbm.at[hhi.at[kk]], hhv.at[slot], sem),
                pltpu.async_copy(ent_hbm.at[thi.at[kk]], thv.at[slot], sem),
            ]

        lanes = lax.iota(jnp.int32, 16)
        dnums = lax.GatherDimensionNumbers(
            offset_dims=(), collapsed_slice_dims=(0,), start_index_map=(0,))

        def take16(x, idx):
            return lax.gather(x, idx[:, None], dnums, (1,),
                              mode=lax.GatherScatterMode.PROMISE_IN_BOUNDS)

        def compute(kk):
            slot = kk % 2

            def body(g, carry):
                vec = jnp.zeros((16,), jnp.float32)
                for l in range(16):
                    i = g * 16 + l
                    acc = None
                    for j in range(D // 16):
                        s = pl.ds(16 * j, 16)
                        r = rv[slot, i, s]
                        a = jnp.abs(hv[slot, i, s] + r - tv[slot, i, s])
                        b = jnp.abs(hhv[slot, i, s] + r - thv[slot, i, s])
                        d = a - b
                        acc = d if acc is None else acc + d
                    t = acc
                    for sh in (1, 2, 4, 8):
                        t = t + take16(t, lanes ^ sh)
                    vec = jnp.where(lanes == l, t, vec)
                out_v[pl.ds(kk * CH + g * 16, 16)] = jnp.maximum(MARGIN + vec, 0.0)
                return carry

            lax.fori_loop(0, CH // 16, body, 0)

        copies = fire(0)
        for kk in range(NCH):
            nxt = fire(kk + 1) if kk + 1 < NCH else None
            for c in copies:
                c.wait()
            compute(kk)
            copies = nxt

        pltpu.sync_copy(out_v, out_hbm.at[wid])

    return k(heads, relations, tails, h_hat, t_hat, ent, rel)


def kernel(heads, relations, tails, h_hat, t_hat, entity_embedding, rel_embedding):
    B = heads.shape[0]
    CB = B // NW
    r3 = lambda x: x.reshape(NW, CB // CH, CH)
    out = _transe_sc(r3(heads), r3(relations), r3(tails), r3(h_hat), r3(t_hat),
                     entity_embedding, rel_embedding, B=B)
    return out.reshape(B, 1)


# speedup vs baseline: 1.3754x; 1.3754x over previous
import functools

import jax
import jax.numpy as jnp
from jax import lax
from jax.experimental import pallas as pl
from jax.experimental.pallas import tpu as pltpu
from jax.experimental.pallas import tpu_sc as plsc

D = 64
MARGIN = 2.0
NC = 2
NS = 16
NW = NC * NS
CH = 64


@functools.partial(jax.jit, static_argnames=("B",))
def _transe_sc(heads, relations, tails, h_hat, t_hat, ent, rel, *, B):
    CB = B // NW
    NCH = CB // CH

    mesh = plsc.VectorSubcoreMesh(core_axis_name="c", subcore_axis_name="s",
                                  num_cores=NC, num_subcores=NS)

    idx_buf = pltpu.VMEM((NCH, CH), jnp.int32)
    row_buf = pltpu.VMEM((2, CH, D), jnp.float32)

    @functools.partial(
        pl.kernel,
        out_type=jax.ShapeDtypeStruct((NW, CB), jnp.float32),
        mesh=mesh,
        scratch_types=[
            idx_buf, idx_buf, idx_buf, idx_buf, idx_buf,
            row_buf, row_buf, row_buf, row_buf, row_buf,
            pltpu.VMEM((CB,), jnp.float32),
            pltpu.SemaphoreType.DMA,
            pltpu.SemaphoreType.DMA,
        ],
        compiler_params=pltpu.CompilerParams(needs_layout_passes=False),
    )
    def k(h_hbm, r_hbm, t_hbm, hh_hbm, th_hbm, ent_hbm, rel_hbm, out_hbm,
          hi, ri, ti, hhi, thi, hv, tv, hhv, thv, rv, out_v,
          sem0, sem1):
        wid = lax.axis_index("s") * NC + lax.axis_index("c")

        pltpu.sync_copy(h_hbm.at[wid], hi)
        pltpu.sync_copy(r_hbm.at[wid], ri)
        pltpu.sync_copy(t_hbm.at[wid], ti)
        pltpu.sync_copy(hh_hbm.at[wid], hhi)
        pltpu.sync_copy(th_hbm.at[wid], thi)

        sems = (sem0, sem1)

        def fire(kk, slot):
            sem = sems[slot]
            for g in range(CH // 16):
                hvec = hi[kk, pl.ds(g * 16, 16)]
                tvec = ti[kk, pl.ds(g * 16, 16)]
                hhvec = hhi[kk, pl.ds(g * 16, 16)]
                thvec = thi[kk, pl.ds(g * 16, 16)]
                rvec = ri[kk, pl.ds(g * 16, 16)]
                for j in range(16):
                    l = g * 16 + j
                    pltpu.async_copy(ent_hbm.at[hvec[j]], hv.at[slot, l], sem)
                    pltpu.async_copy(ent_hbm.at[tvec[j]], tv.at[slot, l], sem)
                    pltpu.async_copy(ent_hbm.at[hhvec[j]], hhv.at[slot, l], sem)
                    pltpu.async_copy(ent_hbm.at[thvec[j]], thv.at[slot, l], sem)
                    pltpu.async_copy(rel_hbm.at[rvec[j]], rv.at[slot, l], sem)

        def drain(slot):
            sem = sems[slot]
            for buf in (hv, tv, hhv, thv, rv):
                pltpu.make_async_copy(ent_hbm.at[pl.ds(0, CH)], buf.at[slot],
                                      sem).wait()

        lanes = lax.iota(jnp.int32, 16)

        def compute(kk, slot):
            for g in range(CH // 16):
                el = lanes + g * 16
                slotv = jnp.full((16,), slot, jnp.int32)

                def dbody(d, acc):
                    dv = jnp.full((16,), d, jnp.int32)
                    h = plsc.load_gather(hv, [slotv, el, dv])
                    t = plsc.load_gather(tv, [slotv, el, dv])
                    hh = plsc.load_gather(hhv, [slotv, el, dv])
                    th = plsc.load_gather(thv, [slotv, el, dv])
                    r = plsc.load_gather(rv, [slotv, el, dv])
                    return acc + (jnp.abs(h + r - t) - jnp.abs(hh + r - th))

                acc = lax.fori_loop(0, D, dbody, jnp.zeros((16,), jnp.float32),
                                    unroll=4)
                out_v[pl.ds(kk * CH + g * 16, 16)] = (
                    jnp.maximum(MARGIN + acc, 0.0))

        fire(0, 0)

        def pair_body(p, carry):
            kk0 = 2 * p

            fire(kk0 + 1, 1)
            drain(0)
            compute(kk0, 0)

            @pl.when(kk0 + 2 < NCH)
            def _():
                fire(kk0 + 2, 0)

            drain(1)
            compute(kk0 + 1, 1)
            return carry

        lax.fori_loop(0, NCH // 2, pair_body, 0)

        pltpu.sync_copy(out_v, out_hbm.at[wid])

    return k(heads, relations, tails, h_hat, t_hat, ent, rel)


def kernel(heads, relations, tails, h_hat, t_hat, entity_embedding, rel_embedding):
    B = heads.shape[0]
    CB = B // NW
    NCH = CB // CH

    shp = (NW, NCH, CH)
    r3 = lambda x: x.reshape(shp)
    out = _transe_sc(r3(heads), r3(relations), r3(tails), r3(h_hat), r3(t_hat),
                     entity_embedding, rel_embedding, B=B)
    return out.reshape(B, 1)
